# baseline (speedup 1.0000x reference)
import functools

import jax
import jax.numpy as jnp
from jax import lax
from jax.experimental import pallas as pl
from jax.experimental.pallas import tpu as pltpu
from jax.experimental.pallas import tpu_sc as plsc

N = 10000
NPAD = 10240
E = 320000
EP = E + N
EPAD1 = 331776
EPADB = 323584
C = 128
CA = 64
CB = C - CA
TENC = 32
EDIM = 48
BLK = 64
NT = 16
NCORE = 2
L = 16

_SC_PARAMS = pltpu.CompilerParams(use_tc_tiling_on_sc=False,
                                  needs_layout_passes=False)


def _mesh():
    return plsc.VectorSubcoreMesh(core_axis_name="c", subcore_axis_name="s")


def _rel_t_kernel():
    nblk = EPADB // (NCORE * NT) // BLK

    @functools.partial(
        pl.kernel,
        out_type=jax.ShapeDtypeStruct((EPADB,), jnp.float32),
        mesh=_mesh(),
        scratch_types=[
            pltpu.VMEM((N,), jnp.float32),
            pltpu.VMEM((BLK,), jnp.int32),
            pltpu.VMEM((BLK,), jnp.float32),
            pltpu.VMEM((BLK,), jnp.float32),
        ],
        compiler_params=_SC_PARAMS,
    )
    def k(lu, srcp, tp, out, lu_v, idx_v, t_v, rel_v):
        c = lax.axis_index("c")
        s = lax.axis_index("s")
        base = (c * NT + s) * (nblk * BLK)
        pltpu.sync_copy(lu, lu_v)

        def blk(b, carry):
            off = base + b * BLK
            pltpu.sync_copy(srcp.at[pl.ds(off, BLK)], idx_v)
            pltpu.sync_copy(tp.at[pl.ds(off, BLK)], t_v)
            for g in range(BLK // L):
                vals = plsc.load_gather(lu_v, [idx_v[pl.ds(g * L, L)]])
                rel_v[pl.ds(g * L, L)] = vals - t_v[pl.ds(g * L, L)]
            pltpu.sync_copy(rel_v, out.at[pl.ds(off, BLK)])
            return carry

        lax.fori_loop(0, nblk, blk, None)

    return k


def _attr_scatter_kernel():
    nblk = EPADB // (NCORE * NT) // BLK
    npt = NPAD // NT

    @functools.partial(
        pl.kernel,
        out_type=(
            jax.ShapeDtypeStruct((NCORE, NPAD, EDIM), jnp.float32),
            jax.ShapeDtypeStruct((NCORE, NT, NPAD // L, L), jnp.float32),
        ),
        mesh=_mesh(),
        scratch_types=[
            pltpu.VMEM_SHARED((NPAD, EDIM), jnp.float32),
            pltpu.VMEM((NPAD // L, L), jnp.float32),
            pltpu.VMEM((BLK,), jnp.int32),
            pltpu.VMEM((BLK, EDIM), jnp.float32),
        ],
        compiler_params=_SC_PARAMS,
    )
    def k(dstp, attr, attr_sum, deg, acc, deg_t, dstv, rows):
        c = lax.axis_index("c")
        s = lax.axis_index("s")
        base = (c * NT + s) * (nblk * BLK)
        z16 = jnp.zeros((L,), jnp.float32)
        ones16 = jnp.ones((L,), jnp.float32)

        def zrow(r, carry):
            for kk in range(EDIM // L):
                rows[r, pl.ds(kk * L, L)] = z16
            return carry

        lax.fori_loop(0, BLK, zrow, None)

        def zdeg(i, carry):
            deg_t[i, :] = z16
            return carry

        lax.fori_loop(0, NPAD // L, zdeg, None)

        def zacc(i, carry):
            pltpu.sync_copy(rows, acc.at[pl.ds(s * npt + i * BLK, BLK)])
            return carry

        lax.fori_loop(0, npt // BLK, zacc, None)
        plsc.subcore_barrier()

        def blk(b, carry):
            off = base + b * BLK
            pltpu.sync_copy(dstp.at[pl.ds(off, BLK)], dstv)
            pltpu.sync_copy(attr.at[pl.ds(off, BLK)], rows)
            pltpu.sync_copy(rows, acc.at[dstv], add=True)
            for g in range(BLK // L):
                dv = dstv[pl.ds(g * L, L)]
                plsc.addupdate_scatter(deg_t, [dv >> 4, dv & 15], ones16)
            return carry

        lax.fori_loop(0, nblk, blk, None)
        plsc.subcore_barrier()

        def wout(i, carry):
            r0 = s * npt + i * BLK
            pltpu.sync_copy(acc.at[pl.ds(r0, BLK)], attr_sum.at[c, pl.ds(r0, BLK)])
            return carry

        lax.fori_loop(0, npt // BLK, wout, None)
        pltpu.sync_copy(deg_t, deg.at[c, s])

    return k


def _edge_logits_kernel(H):
    if H == 2:
        nblk = EPAD1 // NT // BLK
    else:
        nblk = EPAD1 // (NCORE * NT) // BLK
    npt = NPAD // NT
    NK = C // L
    NA = CA // L

    @functools.partial(
        pl.kernel,
        out_type=(
            jax.ShapeDtypeStruct((NCORE, NPAD, CA), jnp.float32),
            jax.ShapeDtypeStruct((NCORE, NT, NPAD // L, L), jnp.float32),
            jax.ShapeDtypeStruct((NCORE if H == 2 else 1, EPAD1), jnp.float32),
        ),
        mesh=_mesh(),
        scratch_types=[
            pltpu.VMEM_SHARED((NPAD, CA), jnp.float32),
            pltpu.VMEM((NPAD // L, L), jnp.float32),
            [pltpu.VMEM((BLK,), jnp.int32) for _ in range(2)],
            [pltpu.VMEM((BLK,), jnp.int32) for _ in range(2)],
            [pltpu.VMEM((BLK,), jnp.int32) for _ in range(2)],
            [pltpu.VMEM((BLK,), jnp.int32) for _ in range(2)],
            [pltpu.VMEM((BLK, C), jnp.float32) for _ in range(2)],
            [pltpu.VMEM((BLK, C), jnp.float32) for _ in range(2)],
            pltpu.VMEM((BLK, C), jnp.float32),
            pltpu.VMEM((BLK, CA), jnp.float32),
            pltpu.VMEM((BLK,), jnp.float32),
            pltpu.VMEM((H, C), jnp.float32),
            [pltpu.SemaphoreType.DMA for _ in range(2)],
            [pltpu.SemaphoreType.DMA for _ in range(2)],
        ],
        compiler_params=_SC_PARAMS,
    )
    def k(srcp, dstp, xl, xr, ef, att, num, den, w_out,
          acc, den_t, srcv, dstv, gidx, didx, gl, gr, eb, obuf, wbuf, attb,
          semg, semr):
        c = lax.axis_index("c")
        s = lax.axis_index("s")
        if H == 2:
            base = s * (nblk * BLK)
            e_off = c * EPAD1
        else:
            base = (c * NT + s) * (nblk * BLK)
            e_off = 0
        z16 = jnp.zeros((L,), jnp.float32)

        pltpu.sync_copy(att, attb)

        def zrow(r, carry):
            for kk in range(NA):
                obuf[r, pl.ds(kk * L, L)] = z16
            return carry

        lax.fori_loop(0, BLK, zrow, None)

        def zdeg(i, carry):
            den_t[i, :] = z16
            return carry

        lax.fori_loop(0, NPAD // L, zdeg, None)

        def zacc(i, carry):
            pltpu.sync_copy(obuf, acc.at[pl.ds(s * npt + i * BLK, BLK)])
            return carry

        lax.fori_loop(0, npt // BLK, zacc, None)
        plsc.subcore_barrier()

        def issue(b, p):
            off = base + b * BLK
            pltpu.sync_copy(srcp.at[pl.ds(off, BLK)], srcv[p])
            pltpu.sync_copy(dstp.at[pl.ds(off, BLK)], dstv[p])
            if H == 2:
                offv = jnp.broadcast_to(c * NPAD, (L,)).astype(jnp.int32)
                for g in range(BLK // L):
                    gidx[p][pl.ds(g * L, L)] = srcv[p][pl.ds(g * L, L)] + offv
                    didx[p][pl.ds(g * L, L)] = dstv[p][pl.ds(g * L, L)] + offv
                gsrc, gdst = gidx[p], didx[p]
            else:
                gsrc, gdst = srcv[p], dstv[p]
            pltpu.async_copy(xl.at[gsrc], gl[p], semg[p])
            pltpu.async_copy(xr.at[gdst], gr[p], semr[p])

        def compute(b, p):
            gsrc = gidx[p] if H == 2 else srcv[p]
            gdst = didx[p] if H == 2 else dstv[p]
            off = base + b * BLK
            pltpu.sync_copy(ef.at[pl.ds(e_off + off, BLK)], eb)
            pltpu.make_async_copy(xl.at[gsrc], gl[p], semg[p]).wait()
            pltpu.make_async_copy(xr.at[gdst], gr[p], semr[p]).wait()

            hrow = c if H == 2 else 0
            attk = [attb[hrow, pl.ds(kk * L, L)] for kk in range(NK)]
            iot = lax.iota(jnp.int32, L)

            def grp(g, carry):
                r0 = g * L
                wacc = z16
                for j in range(L):
                    r = r0 + j
                    glk = [gl[p][r, pl.ds(kk * L, L)] for kk in range(NK)]
                    accv = None
                    for kk in range(NK):
                        u = (glk[kk] + gr[p][r, pl.ds(kk * L, L)]
                             + eb[r, pl.ds(kk * L, L)])
                        lr = jnp.maximum(u, 0.2 * u)
                        term = lr * attk[kk]
                        accv = term if accv is None else accv + term
                    tot = jnp.sum(accv)
                    wv = jnp.exp(jnp.broadcast_to(tot, (L,)))
                    for kk in range(NA):
                        obuf[r, pl.ds(kk * L, L)] = wv * glk[kk]
                    wacc = jnp.where(iot == j, wv, wacc)
                wbuf[pl.ds(r0, L)] = wacc
                dv = dstv[p][pl.ds(r0, L)]
                plsc.addupdate_scatter(den_t, [dv >> 4, dv & 15], wacc)
                return carry

            lax.fori_loop(0, BLK // L, grp, None)
            pltpu.sync_copy(obuf, acc.at[dstv[p]], add=True)
            if H == 2:
                pltpu.sync_copy(wbuf, w_out.at[c, pl.ds(off, BLK)])
            else:
                pltpu.sync_copy(wbuf, w_out.at[0, pl.ds(off, BLK)])

        issue(0, 0)

        def blk_body(i, carry):
            b0 = 2 * i
            b1 = 2 * i + 1

            @pl.when(b1 < nblk)
            def _():
                issue(b1, 1)

            compute(b0, 0)

            @pl.when(b1 + 1 < nblk)
            def _():
                issue(b1 + 1, 0)

            @pl.when(b1 < nblk)
            def _():
                compute(b1, 1)

            return carry

        lax.fori_loop(0, (nblk + 1) // 2, blk_body, None)
        plsc.subcore_barrier()

        def wout(i, carry):
            r0 = s * npt + i * BLK
            pltpu.sync_copy(acc.at[pl.ds(r0, BLK)], num.at[c, pl.ds(r0, BLK)])
            return carry

        lax.fori_loop(0, npt // BLK, wout, None)
        pltpu.sync_copy(den_t, den.at[c, s])

    return k


def _edge_apply_kernel(H):
    if H == 2:
        nblk = EPAD1 // NT // BLK
    else:
        nblk = EPAD1 // (NCORE * NT) // BLK
    npt = NPAD // NT
    NB = CB // L

    @functools.partial(
        pl.kernel,
        out_type=jax.ShapeDtypeStruct((NCORE, NPAD, CB), jnp.float32),
        mesh=_mesh(),
        scratch_types=[
            pltpu.VMEM_SHARED((NPAD, CB), jnp.float32),
            [pltpu.VMEM((BLK,), jnp.int32) for _ in range(2)],
            [pltpu.VMEM((BLK,), jnp.int32) for _ in range(2)],
            [pltpu.VMEM((BLK,), jnp.int32) for _ in range(2)],
            [pltpu.VMEM((BLK, CB), jnp.float32) for _ in range(2)],
            [pltpu.VMEM((BLK,), jnp.float32) for _ in range(2)],
            [pltpu.SemaphoreType.DMA for _ in range(2)],
        ],
        compiler_params=_SC_PARAMS,
    )
    def k(srcp, dstp, xlB, w_in, num, acc, srcv, dstv, gidx, glB, wbuf, sem):
        c = lax.axis_index("c")
        s = lax.axis_index("s")
        if H == 2:
            base = s * (nblk * BLK)
        else:
            base = (c * NT + s) * (nblk * BLK)
        z16 = jnp.zeros((L,), jnp.float32)

        def zrow(r, carry):
            for kk in range(NB):
                glB[0][r, pl.ds(kk * L, L)] = z16
            return carry

        lax.fori_loop(0, BLK, zrow, None)

        def zacc(i, carry):
            pltpu.sync_copy(glB[0], acc.at[pl.ds(s * npt + i * BLK, BLK)])
            return carry

        lax.fori_loop(0, npt // BLK, zacc, None)
        plsc.subcore_barrier()

        def issue(b, p):
            off = base + b * BLK
            pltpu.sync_copy(srcp.at[pl.ds(off, BLK)], srcv[p])
            pltpu.sync_copy(dstp.at[pl.ds(off, BLK)], dstv[p])
            if H == 2:
                pltpu.sync_copy(w_in.at[c, pl.ds(off, BLK)], wbuf[p])
                offv = jnp.broadcast_to(c * NPAD, (L,)).astype(jnp.int32)
                for g in range(BLK // L):
                    gidx[p][pl.ds(g * L, L)] = srcv[p][pl.ds(g * L, L)] + offv
                gsrc = gidx[p]
            else:
                pltpu.sync_copy(w_in.at[0, pl.ds(off, BLK)], wbuf[p])
                gsrc = srcv[p]
            pltpu.async_copy(xlB.at[gsrc], glB[p], sem[p])

        def compute(b, p):
            gsrc = gidx[p] if H == 2 else srcv[p]
            pltpu.make_async_copy(xlB.at[gsrc], glB[p], sem[p]).wait()

            def grp(g, carry):
                r0 = g * L
                for j in range(L):
                    r = r0 + j
                    ridx = jnp.broadcast_to(r, (L,)).astype(jnp.int32)
                    wv = plsc.load_gather(wbuf[p], [ridx])
                    for kk in range(NB):
                        glB[p][r, pl.ds(kk * L, L)] = (
                            wv * glB[p][r, pl.ds(kk * L, L)])
                return carry

            lax.fori_loop(0, BLK // L, grp, None)
            pltpu.sync_copy(glB[p], acc.at[dstv[p]], add=True)

        issue(0, 0)

        def blk_body(i, carry):
            b0 = 2 * i
            b1 = 2 * i + 1

            @pl.when(b1 < nblk)
            def _():
                issue(b1, 1)

            compute(b0, 0)

            @pl.when(b1 + 1 < nblk)
            def _():
                issue(b1 + 1, 0)

            @pl.when(b1 < nblk)
            def _():
                compute(b1, 1)

            return carry

        lax.fori_loop(0, (nblk + 1) // 2, blk_body, None)
        plsc.subcore_barrier()

        def wout(i, carry):
            r0 = s * npt + i * BLK
            pltpu.sync_copy(acc.at[pl.ds(r0, BLK)], num.at[c, pl.ds(r0, BLK)])
            return carry

        lax.fori_loop(0, npt // BLK, wout, None)

    return k


_REL_K = _rel_t_kernel()
_ATTR_K = _attr_scatter_kernel()
_LOGITS_K2 = _edge_logits_kernel(2)
_LOGITS_K1 = _edge_logits_kernel(1)
_APPLY_K2 = _edge_apply_kernel(2)
_APPLY_K1 = _edge_apply_kernel(1)


def _mm_headed(A, W, H, blk_rows):
    M, K = A.shape

    def kern(a_ref, w_ref, o_ref):
        o_ref[0] = jnp.dot(a_ref[...], w_ref[...],
                           preferred_element_type=jnp.float32)

    return pl.pallas_call(
        kern,
        grid=(H, M // blk_rows),
        in_specs=[
            pl.BlockSpec((blk_rows, K), lambda h, i: (i, 0)),
            pl.BlockSpec((K, 128), lambda h, i: (0, h)),
        ],
        out_specs=pl.BlockSpec((1, blk_rows, 128), lambda h, i: (h, i, 0)),
        out_shape=jax.ShapeDtypeStruct((H, M, 128), jnp.float32),
    )(A, W)


def _edge_attr(rel_col, msg, Wt, bt):
    blk = 3200

    def kern(r_ref, m_ref, wt_ref, bt_ref, o_ref):
        enc = jnp.cos(r_ref[...] * wt_ref[...] + bt_ref[...])
        o_ref[...] = jnp.concatenate([enc, m_ref[...]], axis=1)

    return pl.pallas_call(
        kern,
        grid=(E // blk,),
        in_specs=[
            pl.BlockSpec((blk, 1), lambda i: (i, 0)),
            pl.BlockSpec((blk, 16), lambda i: (i, 0)),
            pl.BlockSpec((1, TENC), lambda i: (0, 0)),
            pl.BlockSpec((1, TENC), lambda i: (0, 0)),
        ],
        out_specs=pl.BlockSpec((blk, EDIM), lambda i: (i, 0)),
        out_shape=jax.ShapeDtypeStruct((E, EDIM), jnp.float32),
    )(rel_col, msg, Wt, bt)


def _attr_mean(attr_sum, deg):
    blk = 1280

    def kern(a_ref, d_ref, o_ref):
        asum = a_ref[0] + a_ref[1]
        dsum = jnp.sum(d_ref[...], axis=(0, 1))
        o_ref[...] = asum / jnp.clip(dsum, 1.0, None)[:, None]

    return pl.pallas_call(
        kern,
        grid=(NPAD // blk,),
        in_specs=[
            pl.BlockSpec((2, blk, EDIM), lambda i: (0, i, 0)),
            pl.BlockSpec((2, NT, blk), lambda i: (0, 0, i)),
        ],
        out_specs=pl.BlockSpec((blk, EDIM), lambda i: (i, 0)),
        out_shape=jax.ShapeDtypeStruct((NPAD, EDIM), jnp.float32),
    )(attr_sum, deg)


def _div1(numA, numB, den, b1):
    blk = 1280

    def kern(na_ref, nb_ref, d_ref, b_ref, o_ref):
        i = pl.program_id(0)
        de = jnp.sum(d_ref[...], axis=1)
        de = jnp.where(de == 0.0, 1.0, de)
        h0 = jnp.concatenate([na_ref[0], nb_ref[0]], axis=1) / de[0][:, None]
        h1 = jnp.concatenate([na_ref[1], nb_ref[1]], axis=1) / de[1][:, None]
        h = jnp.concatenate([h0, h1], axis=1) + b_ref[...]
        h = jnp.maximum(h, 0.0)
        grow = i * blk + lax.broadcasted_iota(jnp.int32, (blk, 1), 0)
        o_ref[...] = jnp.where(grow < N, h, 0.0)

    return pl.pallas_call(
        kern,
        grid=(NPAD // blk,),
        in_specs=[
            pl.BlockSpec((2, blk, CA), lambda i: (0, i, 0)),
            pl.BlockSpec((2, blk, CB), lambda i: (0, i, 0)),
            pl.BlockSpec((2, NT, blk), lambda i: (0, 0, i)),
            pl.BlockSpec((1, 2 * C), lambda i: (0, 0)),
        ],
        out_specs=pl.BlockSpec((blk, 2 * C), lambda i: (i, 0)),
        out_shape=jax.ShapeDtypeStruct((NPAD, 2 * C), jnp.float32),
    )(numA, numB, den, b1)


def _div2(numA, numB, den, b2):
    blk = 1280

    def kern(na_ref, nb_ref, d_ref, b_ref, o_ref):
        de = jnp.sum(d_ref[...], axis=(0, 1))
        de = jnp.where(de == 0.0, 1.0, de)
        nsum = jnp.concatenate([na_ref[0] + na_ref[1], nb_ref[0] + nb_ref[1]],
                               axis=1)
        o_ref[...] = nsum / de[:, None] + b_ref[...]

    return pl.pallas_call(
        kern,
        grid=(NPAD // blk,),
        in_specs=[
            pl.BlockSpec((2, blk, CA), lambda i: (0, i, 0)),
            pl.BlockSpec((2, blk, CB), lambda i: (0, i, 0)),
            pl.BlockSpec((2, NT, blk), lambda i: (0, 0, i)),
            pl.BlockSpec((1, C), lambda i: (0, 0)),
        ],
        out_specs=pl.BlockSpec((blk, C), lambda i: (i, 0)),
        out_shape=jax.ShapeDtypeStruct((NPAD, C), jnp.float32),
    )(numA, numB, den, b2)


def kernel(x, last_update, edge_index, t, msg, Wt, bt,
           Wl1, Wr1, We1, att1, b1, Wl2, Wr2, We2, att2, b2):
    f32 = jnp.float32
    i32 = jnp.int32
    src = edge_index[0].astype(i32)
    dst = edge_index[1].astype(i32)

    srcA = jnp.concatenate([src, jnp.zeros((EPADB - E,), i32)])
    tA = jnp.concatenate([t, jnp.zeros((EPADB - E,), f32)])
    rel = _REL_K(last_update, srcA, tA)
    rel_col = rel[:E, None]

    ea = _edge_attr(rel_col, msg, Wt, bt.reshape(1, TENC))

    dstB = jnp.concatenate([dst, jnp.full((EPADB - E,), N, i32)])
    eaB = jnp.concatenate([ea, jnp.zeros((EPADB - E, EDIM), f32)])
    attr_sum, degB = _ATTR_K(dstB, eaB)
    attr_mean = _attr_mean(attr_sum, degB.reshape(NCORE, NT, NPAD))

    loop_idx = jnp.arange(N, dtype=i32)
    padE = EPAD1 - EP
    src2 = jnp.concatenate([src, loop_idx, jnp.zeros((padE,), i32)])
    dst2 = jnp.concatenate([dst, loop_idx, jnp.full((padE,), N, i32)])
    ea_full = jnp.concatenate([ea, attr_mean[:N], jnp.zeros((padE, EDIM), f32)])

    x_pad = jnp.pad(x, ((0, NPAD - N), (0, 0)))
    xl1 = _mm_headed(x_pad, Wl1, 2, 1280).reshape(2 * NPAD, C)
    xr1 = _mm_headed(x_pad, Wr1, 2, 1280).reshape(2 * NPAD, C)
    e1 = _mm_headed(ea_full, We1, 2, 4096).reshape(2 * EPAD1, C)
    xl1B = xl1[:, CA:]

    numA1, den1, w1 = _LOGITS_K2(src2, dst2, xl1, xr1, e1, att1)
    numB1 = _APPLY_K2(src2, dst2, xl1B, w1)
    h = _div1(numA1, numB1, den1.reshape(NCORE, NT, NPAD), b1.reshape(1, 2 * C))

    xl2 = _mm_headed(h, Wl2, 1, 1280).reshape(NPAD, C)
    xr2 = _mm_headed(h, Wr2, 1, 1280).reshape(NPAD, C)
    e2 = _mm_headed(ea_full, We2, 1, 4096).reshape(EPAD1, C)
    xl2B = xl2[:, CA:]

    numA2, den2, w2 = _LOGITS_K1(src2, dst2, xl2, xr2, e2, att2)
    numB2 = _APPLY_K1(src2, dst2, xl2B, w2)
    out = _div2(numA2, numB2, den2.reshape(NCORE, NT, NPAD), b2.reshape(1, C))
    return out[:N]


# speedup vs baseline: 1.2383x; 1.2383x over previous
import functools

import jax
import jax.numpy as jnp
from jax import lax
from jax.experimental import pallas as pl
from jax.experimental.pallas import tpu as pltpu
from jax.experimental.pallas import tpu_sc as plsc

N = 10000
NPAD = 10240
E = 320000
EP = E + N
EPAD1 = 331776
EPADB = 323584
C = 128
CA = 64
CB = C - CA
TENC = 32
EDIM = 48
BLKA = 128
BLK = 96
SB = 12
NBLKT = EPAD1 // BLK
NT = 16
NCORE = 2
L = 16

_SC_PARAMS = pltpu.CompilerParams(use_tc_tiling_on_sc=False,
                                  needs_layout_passes=False)


def _mesh():
    return plsc.VectorSubcoreMesh(core_axis_name="c", subcore_axis_name="s")


def _rel_t_kernel():
    nblk = EPADB // (NCORE * NT) // BLKA

    @functools.partial(
        pl.kernel,
        out_type=jax.ShapeDtypeStruct((EPADB,), jnp.float32),
        mesh=_mesh(),
        scratch_types=[
            pltpu.VMEM((N,), jnp.float32),
            pltpu.VMEM((BLKA,), jnp.int32),
            pltpu.VMEM((BLKA,), jnp.float32),
            pltpu.VMEM((BLKA,), jnp.float32),
        ],
        compiler_params=_SC_PARAMS,
    )
    def k(lu, srcp, tp, out, lu_v, idx_v, t_v, rel_v):
        c = lax.axis_index("c")
        s = lax.axis_index("s")
        base = (c * NT + s) * (nblk * BLKA)
        pltpu.sync_copy(lu, lu_v)

        def blk(b, carry):
            off = base + b * BLKA
            pltpu.sync_copy(srcp.at[pl.ds(off, BLKA)], idx_v)
            pltpu.sync_copy(tp.at[pl.ds(off, BLKA)], t_v)
            for g in range(BLKA // L):
                vals = plsc.load_gather(lu_v, [idx_v[pl.ds(g * L, L)]])
                rel_v[pl.ds(g * L, L)] = vals - t_v[pl.ds(g * L, L)]
            pltpu.sync_copy(rel_v, out.at[pl.ds(off, BLKA)])
            return carry

        lax.fori_loop(0, nblk, blk, None)

    return k


def _attr_scatter_kernel():
    nblk = EPADB // (NCORE * NT) // BLKA
    npt = NPAD // NT

    @functools.partial(
        pl.kernel,
        out_type=(
            jax.ShapeDtypeStruct((NCORE, NPAD, EDIM), jnp.float32),
            jax.ShapeDtypeStruct((NCORE, NT, NPAD // L, L), jnp.float32),
        ),
        mesh=_mesh(),
        scratch_types=[
            pltpu.VMEM_SHARED((NPAD, EDIM), jnp.float32),
            pltpu.VMEM((NPAD // L, L), jnp.float32),
            pltpu.VMEM((BLKA,), jnp.int32),
            pltpu.VMEM((BLKA, EDIM), jnp.float32),
        ],
        compiler_params=_SC_PARAMS,
    )
    def k(dstp, attr, attr_sum, deg, acc, deg_t, dstv, rows):
        c = lax.axis_index("c")
        s = lax.axis_index("s")
        base = (c * NT + s) * (nblk * BLKA)
        z16 = jnp.zeros((L,), jnp.float32)
        ones16 = jnp.ones((L,), jnp.float32)

        def zrow(r, carry):
            for kk in range(EDIM // L):
                rows[r, pl.ds(kk * L, L)] = z16
            return carry

        lax.fori_loop(0, BLKA, zrow, None)

        def zdeg(i, carry):
            deg_t[i, :] = z16
            return carry

        lax.fori_loop(0, NPAD // L, zdeg, None)

        def zacc(i, carry):
            pltpu.sync_copy(rows, acc.at[pl.ds(s * npt + i * BLKA, BLKA)])
            return carry

        lax.fori_loop(0, npt // BLKA, zacc, None)
        plsc.subcore_barrier()

        def blk(b, carry):
            off = base + b * BLKA
            pltpu.sync_copy(dstp.at[pl.ds(off, BLKA)], dstv)
            pltpu.sync_copy(attr.at[pl.ds(off, BLKA)], rows)
            pltpu.sync_copy(rows, acc.at[dstv], add=True)
            for g in range(BLKA // L):
                dv = dstv[pl.ds(g * L, L)]
                plsc.addupdate_scatter(deg_t, [dv >> 4, dv & 15], ones16)
            return carry

        lax.fori_loop(0, nblk, blk, None)
        plsc.subcore_barrier()

        def wout(i, carry):
            r0 = s * npt + i * BLKA
            pltpu.sync_copy(acc.at[pl.ds(r0, BLKA)], attr_sum.at[c, pl.ds(r0, BLKA)])
            return carry

        lax.fori_loop(0, npt // BLKA, wout, None)
        pltpu.sync_copy(deg_t, deg.at[c, s])

    return k


def _acc_writeout(acc, dst3, c, s, src_zero=None):
    npt = NPAD // NT
    for i in range(npt // BLK):
        r0 = s * npt + i * BLK
        if src_zero is not None:
            pltpu.sync_copy(src_zero, acc.at[pl.ds(r0, BLK)])
        else:
            pltpu.sync_copy(acc.at[pl.ds(r0, BLK)], dst3.at[c, pl.ds(r0, BLK)])
    r0 = s * npt + (npt // BLK) * BLK
    tail = npt - (npt // BLK) * BLK
    if src_zero is not None:
        pltpu.sync_copy(src_zero.at[pl.ds(0, tail)], acc.at[pl.ds(r0, tail)])
    else:
        pltpu.sync_copy(acc.at[pl.ds(r0, tail)], dst3.at[c, pl.ds(r0, tail)])


def _edge_logits_kernel(H):
    HN = NCORE if H == 2 else 1
    nblk = NBLKT // NT if H == 2 else NBLKT // (NCORE * NT)
    n_sb = nblk // SB
    NK = C // L
    NA = CA // L
    G = BLK // L

    @functools.partial(
        pl.kernel,
        out_type=(
            jax.ShapeDtypeStruct((NCORE, NPAD, CA), jnp.float32),
            jax.ShapeDtypeStruct((NCORE, NT, NPAD // L, L), jnp.float32),
            (jax.ShapeDtypeStruct((HN, NBLKT, BLK), jnp.float32) if H == 2
             else jax.ShapeDtypeStruct((NBLKT, BLK), jnp.float32)),
        ),
        mesh=_mesh(),
        scratch_types=[
            pltpu.VMEM_SHARED((NPAD, CA), jnp.float32),
            pltpu.VMEM((NPAD // L, L), jnp.float32),
            pltpu.VMEM((SB, BLK), jnp.int32),
            pltpu.VMEM((SB, BLK), jnp.int32),
            pltpu.VMEM((SB, BLK), jnp.int32),
            [pltpu.VMEM((BLK, C), jnp.float32) for _ in range(2)],
            [pltpu.VMEM((BLK, C), jnp.float32) for _ in range(2)],
            pltpu.VMEM((BLK, C), jnp.float32),
            [pltpu.VMEM((BLK, CA), jnp.float32) for _ in range(2)],
            pltpu.VMEM((SB, BLK), jnp.float32),
            pltpu.VMEM((H, C), jnp.float32),
            [pltpu.SemaphoreType.DMA for _ in range(2)],
            [pltpu.SemaphoreType.DMA for _ in range(2)],
            [pltpu.SemaphoreType.DMA for _ in range(2)],
            pltpu.SemaphoreType.DMA,
        ],
        compiler_params=_SC_PARAMS,
    )
    def k(srcg, dstg, dstp, xl, xr, ef, att, num, den, w_out,
          acc, den_t, srcg_ch, dstg_ch, dstp_ch, gl, gr, eb, obuf, wchunk,
          attb, semg, semr, semsc, semw):
        c = lax.axis_index("c")
        s = lax.axis_index("s")
        if H == 2:
            base_blk = s * nblk
            e_off = c * EPAD1
            hw = c
        else:
            base_blk = (c * NT + s) * nblk
            e_off = 0
        z16 = jnp.zeros((L,), jnp.float32)

        pltpu.sync_copy(att, attb)

        def zrow(r, carry):
            for kk in range(NA):
                obuf[0][r, pl.ds(kk * L, L)] = z16
            return carry

        lax.fori_loop(0, BLK, zrow, None)

        def zdeg(i, carry):
            den_t[i, :] = z16
            return carry

        lax.fori_loop(0, NPAD // L, zdeg, None)
        _acc_writeout(acc, None, c, s, src_zero=obuf[0])
        plsc.subcore_barrier()

        def issue(jb, p):
            pltpu.async_copy(xl.at[srcg_ch.at[jb]], gl[p], semg[p])
            pltpu.async_copy(xr.at[dstg_ch.at[jb]], gr[p], semr[p])

        def drain_sc(p):
            pltpu.make_async_copy(obuf[p], acc.at[dstp_ch.at[0]],
                                  semsc[p]).wait()

        def drain_w():
            if H == 2:
                pltpu.make_async_copy(wchunk, w_out.at[c, pl.ds(0, SB)],
                                      semw).wait()
            else:
                pltpu.make_async_copy(wchunk, w_out.at[pl.ds(0, SB)],
                                      semw).wait()

        def compute(sb, jp, jb, p):
            eoff = (e_off + (base_blk + sb * SB + jb) * BLK)
            pltpu.sync_copy(ef.at[pl.ds(eoff, BLK)], eb)
            pltpu.make_async_copy(xl.at[srcg_ch.at[jb]], gl[p], semg[p]).wait()
            pltpu.make_async_copy(xr.at[dstg_ch.at[jb]], gr[p], semr[p]).wait()

            @pl.when(jp > 0)
            def _wait_sc():
                drain_sc(p)

            hrow = c if H == 2 else 0
            attk = [attb[hrow, pl.ds(kk * L, L)] for kk in range(NK)]
            iot = lax.iota(jnp.int32, L)

            def grp(g, carry):
                r0 = g * L
                wacc = z16
                for j in range(L):
                    r = r0 + j
                    glk = [gl[p][r, pl.ds(kk * L, L)] for kk in range(NK)]
                    accv = None
                    for kk in range(NK):
                        u = (glk[kk] + gr[p][r, pl.ds(kk * L, L)]
                             + eb[r, pl.ds(kk * L, L)])
                        lr = jnp.maximum(u, 0.2 * u)
                        term = lr * attk[kk]
                        accv = term if accv is None else accv + term
                    tot = jnp.sum(accv)
                    wv = jnp.exp(jnp.broadcast_to(tot, (L,)))
                    for kk in range(NA):
                        obuf[p][r, pl.ds(kk * L, L)] = wv * glk[kk]
                    wacc = jnp.where(iot == j, wv, wacc)
                wchunk[jb, pl.ds(r0, L)] = wacc
                dv = dstp_ch[jb, pl.ds(r0, L)]
                plsc.addupdate_scatter(den_t, [dv >> 4, dv & 15], wacc)
                return carry

            lax.fori_loop(0, G, grp, None)
            pltpu.async_copy(obuf[p], acc.at[dstp_ch.at[jb]], semsc[p],
                             add=True)

        def sb_body(sb, carry):
            @pl.when(sb > 0)
            def _drain_prev():
                drain_sc(0)
                drain_sc(1)
                drain_w()

            cb = base_blk + sb * SB
            if H == 2:
                pltpu.sync_copy(srcg.at[c, pl.ds(cb, SB)], srcg_ch)
                pltpu.sync_copy(dstg.at[c, pl.ds(cb, SB)], dstg_ch)
            else:
                pltpu.sync_copy(srcg.at[pl.ds(cb, SB)], srcg_ch)
                pltpu.sync_copy(dstg.at[pl.ds(cb, SB)], dstg_ch)
            pltpu.sync_copy(dstp.at[pl.ds(cb, SB)], dstp_ch)
            issue(0, 0)

            def pair(jp, carry2):
                jb0 = 2 * jp
                jb1 = 2 * jp + 1
                issue(jb1, 1)
                compute(sb, jp, jb0, 0)

                @pl.when(jb1 + 1 < SB)
                def _issue_next():
                    issue(jb1 + 1, 0)

                compute(sb, jp, jb1, 1)
                return carry2

            lax.fori_loop(0, SB // 2, pair, None)
            if H == 2:
                pltpu.async_copy(wchunk, w_out.at[c, pl.ds(cb, SB)], semw)
            else:
                pltpu.async_copy(wchunk, w_out.at[pl.ds(cb, SB)], semw)
            return carry

        lax.fori_loop(0, n_sb, sb_body, None)
        drain_sc(0)
        drain_sc(1)
        drain_w()
        plsc.subcore_barrier()
        _acc_writeout(acc, num, c, s)
        pltpu.sync_copy(den_t, den.at[c, s])

    return k


def _edge_apply_kernel(H):
    nblk = NBLKT // NT if H == 2 else NBLKT // (NCORE * NT)
    n_sb = nblk // SB
    NB = CB // L
    G = BLK // L

    @functools.partial(
        pl.kernel,
        out_type=jax.ShapeDtypeStruct((NCORE, NPAD, CB), jnp.float32),
        mesh=_mesh(),
        scratch_types=[
            pltpu.VMEM_SHARED((NPAD, CB), jnp.float32),
            pltpu.VMEM((SB, BLK), jnp.int32),
            pltpu.VMEM((SB, BLK), jnp.int32),
            pltpu.VMEM((SB, BLK), jnp.float32),
            [pltpu.VMEM((BLK, CB), jnp.float32) for _ in range(2)],
            [pltpu.VMEM((BLK, CB), jnp.float32) for _ in range(2)],
            [pltpu.SemaphoreType.DMA for _ in range(2)],
            [pltpu.SemaphoreType.DMA for _ in range(2)],
        ],
        compiler_params=_SC_PARAMS,
    )
    def k(srcg, dstp, xlB, w_in, num,
          acc, srcg_ch, dstp_ch, wch, glB, sbuf, semg, semsc):
        c = lax.axis_index("c")
        s = lax.axis_index("s")
        if H == 2:
            base_blk = s * nblk
        else:
            base_blk = (c * NT + s) * nblk
        z16 = jnp.zeros((L,), jnp.float32)

        def zrow(r, carry):
            for kk in range(NB):
                sbuf[0][r, pl.ds(kk * L, L)] = z16
            return carry

        lax.fori_loop(0, BLK, zrow, None)
        _acc_writeout(acc, None, c, s, src_zero=sbuf[0])
        plsc.subcore_barrier()

        def issue(jb, p):
            pltpu.async_copy(xlB.at[srcg_ch.at[jb]], glB[p], semg[p])

        def drain_sc(p):
            pltpu.make_async_copy(sbuf[p], acc.at[dstp_ch.at[0]],
                                  semsc[p]).wait()

        def compute(jp, jb, p):
            pltpu.make_async_copy(xlB.at[srcg_ch.at[jb]], glB[p],
                                  semg[p]).wait()

            @pl.when(jp > 0)
            def _wait_sc():
                drain_sc(p)

            def grp(g, carry):
                r0 = g * L
                for j in range(L):
                    r = r0 + j
                    jbv = jnp.broadcast_to(jb, (L,)).astype(jnp.int32)
                    rv = jnp.broadcast_to(r, (L,)).astype(jnp.int32)
                    wv = plsc.load_gather(wch, [jbv, rv])
                    for kk in range(NB):
                        sbuf[p][r, pl.ds(kk * L, L)] = (
                            wv * glB[p][r, pl.ds(kk * L, L)])
                return carry

            lax.fori_loop(0, G, grp, None)
            pltpu.async_copy(sbuf[p], acc.at[dstp_ch.at[jb]], semsc[p],
                             add=True)

        def sb_body(sb, carry):
            @pl.when(sb > 0)
            def _drain_prev():
                drain_sc(0)
                drain_sc(1)

            cb = base_blk + sb * SB
            if H == 2:
                pltpu.sync_copy(srcg.at[c, pl.ds(cb, SB)], srcg_ch)
                pltpu.sync_copy(w_in.at[c, pl.ds(cb, SB)], wch)
            else:
                pltpu.sync_copy(srcg.at[pl.ds(cb, SB)], srcg_ch)
                pltpu.sync_copy(w_in.at[pl.ds(cb, SB)], wch)
            pltpu.sync_copy(dstp.at[pl.ds(cb, SB)], dstp_ch)
            issue(0, 0)

            def pair(jp, carry2):
                jb0 = 2 * jp
                jb1 = 2 * jp + 1
                issue(jb1, 1)
                compute(jp, jb0, 0)

                @pl.when(jb1 + 1 < SB)
                def _issue_next():
                    issue(jb1 + 1, 0)

                compute(jp, jb1, 1)
                return carry2

            lax.fori_loop(0, SB // 2, pair, None)
            return carry

        lax.fori_loop(0, n_sb, sb_body, None)
        drain_sc(0)
        drain_sc(1)
        plsc.subcore_barrier()
        _acc_writeout(acc, num, c, s)

    return k


_REL_K = _rel_t_kernel()
_ATTR_K = _attr_scatter_kernel()
_LOGITS_K2 = _edge_logits_kernel(2)
_LOGITS_K1 = _edge_logits_kernel(1)
_APPLY_K2 = _edge_apply_kernel(2)
_APPLY_K1 = _edge_apply_kernel(1)


def _mm_headed(A, W, H, blk_rows):
    M, K = A.shape

    def kern(a_ref, w_ref, o_ref):
        o_ref[0] = jnp.dot(a_ref[...], w_ref[...],
                           preferred_element_type=jnp.float32)

    return pl.pallas_call(
        kern,
        grid=(H, M // blk_rows),
        in_specs=[
            pl.BlockSpec((blk_rows, K), lambda h, i: (i, 0)),
            pl.BlockSpec((K, 128), lambda h, i: (0, h)),
        ],
        out_specs=pl.BlockSpec((1, blk_rows, 128), lambda h, i: (h, i, 0)),
        out_shape=jax.ShapeDtypeStruct((H, M, 128), jnp.float32),
    )(A, W)


def _edge_attr(rel_col, msg, Wt, bt):
    blk = 3200

    def kern(r_ref, m_ref, wt_ref, bt_ref, o_ref):
        enc = jnp.cos(r_ref[...] * wt_ref[...] + bt_ref[...])
        o_ref[...] = jnp.concatenate([enc, m_ref[...]], axis=1)

    return pl.pallas_call(
        kern,
        grid=(E // blk,),
        in_specs=[
            pl.BlockSpec((blk, 1), lambda i: (i, 0)),
            pl.BlockSpec((blk, 16), lambda i: (i, 0)),
            pl.BlockSpec((1, TENC), lambda i: (0, 0)),
            pl.BlockSpec((1, TENC), lambda i: (0, 0)),
        ],
        out_specs=pl.BlockSpec((blk, EDIM), lambda i: (i, 0)),
        out_shape=jax.ShapeDtypeStruct((E, EDIM), jnp.float32),
    )(rel_col, msg, Wt, bt)


def _attr_mean(attr_sum, deg):
    blk = 1280

    def kern(a_ref, d_ref, o_ref):
        asum = a_ref[0] + a_ref[1]
        dsum = jnp.sum(d_ref[...], axis=(0, 1))
        o_ref[...] = asum / jnp.clip(dsum, 1.0, None)[:, None]

    return pl.pallas_call(
        kern,
        grid=(NPAD // blk,),
        in_specs=[
            pl.BlockSpec((2, blk, EDIM), lambda i: (0, i, 0)),
            pl.BlockSpec((2, NT, blk), lambda i: (0, 0, i)),
        ],
        out_specs=pl.BlockSpec((blk, EDIM), lambda i: (i, 0)),
        out_shape=jax.ShapeDtypeStruct((NPAD, EDIM), jnp.float32),
    )(attr_sum, deg)


def _div1(numA, numB, den, b1):
    blk = 1280

    def kern(na_ref, nb_ref, d_ref, b_ref, o_ref):
        i = pl.program_id(0)
        de = jnp.sum(d_ref[...], axis=1)
        de = jnp.where(de == 0.0, 1.0, de)
        h0 = jnp.concatenate([na_ref[0], nb_ref[0]], axis=1) / de[0][:, None]
        h1 = jnp.concatenate([na_ref[1], nb_ref[1]], axis=1) / de[1][:, None]
        h = jnp.concatenate([h0, h1], axis=1) + b_ref[...]
        h = jnp.maximum(h, 0.0)
        grow = i * blk + lax.broadcasted_iota(jnp.int32, (blk, 1), 0)
        o_ref[...] = jnp.where(grow < N, h, 0.0)

    return pl.pallas_call(
        kern,
        grid=(NPAD // blk,),
        in_specs=[
            pl.BlockSpec((2, blk, CA), lambda i: (0, i, 0)),
            pl.BlockSpec((2, blk, CB), lambda i: (0, i, 0)),
            pl.BlockSpec((2, NT, blk), lambda i: (0, 0, i)),
            pl.BlockSpec((1, 2 * C), lambda i: (0, 0)),
        ],
        out_specs=pl.BlockSpec((blk, 2 * C), lambda i: (i, 0)),
        out_shape=jax.ShapeDtypeStruct((NPAD, 2 * C), jnp.float32),
    )(numA, numB, den, b1)


def _div2(numA, numB, den, b2):
    blk = 1280

    def kern(na_ref, nb_ref, d_ref, b_ref, o_ref):
        de = jnp.sum(d_ref[...], axis=(0, 1))
        de = jnp.where(de == 0.0, 1.0, de)
        nsum = jnp.concatenate([na_ref[0] + na_ref[1], nb_ref[0] + nb_ref[1]],
                               axis=1)
        o_ref[...] = nsum / de[:, None] + b_ref[...]

    return pl.pallas_call(
        kern,
        grid=(NPAD // blk,),
        in_specs=[
            pl.BlockSpec((2, blk, CA), lambda i: (0, i, 0)),
            pl.BlockSpec((2, blk, CB), lambda i: (0, i, 0)),
            pl.BlockSpec((2, NT, blk), lambda i: (0, 0, i)),
            pl.BlockSpec((1, C), lambda i: (0, 0)),
        ],
        out_specs=pl.BlockSpec((blk, C), lambda i: (i, 0)),
        out_shape=jax.ShapeDtypeStruct((NPAD, C), jnp.float32),
    )(numA, numB, den, b2)


def kernel(x, last_update, edge_index, t, msg, Wt, bt,
           Wl1, Wr1, We1, att1, b1, Wl2, Wr2, We2, att2, b2):
    f32 = jnp.float32
    i32 = jnp.int32
    src = edge_index[0].astype(i32)
    dst = edge_index[1].astype(i32)

    srcA = jnp.concatenate([src, jnp.zeros((EPADB - E,), i32)])
    tA = jnp.concatenate([t, jnp.zeros((EPADB - E,), f32)])
    rel = _REL_K(last_update, srcA, tA)
    rel_col = rel[:E, None]

    ea = _edge_attr(rel_col, msg, Wt, bt.reshape(1, TENC))

    dstB = jnp.concatenate([dst, jnp.full((EPADB - E,), N, i32)])
    eaB = jnp.concatenate([ea, jnp.zeros((EPADB - E, EDIM), f32)])
    attr_sum, degB = _ATTR_K(dstB, eaB)
    attr_mean = _attr_mean(attr_sum, degB.reshape(NCORE, NT, NPAD))

    loop_idx = jnp.arange(N, dtype=i32)
    padE = EPAD1 - EP
    src2 = jnp.concatenate([src, loop_idx, jnp.zeros((padE,), i32)])
    dst2 = jnp.concatenate([dst, loop_idx, jnp.full((padE,), N, i32)])
    ea_full = jnp.concatenate([ea, attr_mean[:N], jnp.zeros((padE, EDIM), f32)])

    src2g2 = jnp.stack([src2, src2 + NPAD]).reshape(2, NBLKT, BLK)
    dst2g2 = jnp.stack([dst2, dst2 + NPAD]).reshape(2, NBLKT, BLK)
    src2g1 = src2.reshape(NBLKT, BLK)
    dst2g1 = dst2.reshape(NBLKT, BLK)
    dst2p = dst2.reshape(NBLKT, BLK)

    x_pad = jnp.pad(x, ((0, NPAD - N), (0, 0)))
    xl1 = _mm_headed(x_pad, Wl1, 2, 1280).reshape(2 * NPAD, C)
    xr1 = _mm_headed(x_pad, Wr1, 2, 1280).reshape(2 * NPAD, C)
    e1 = _mm_headed(ea_full, We1, 2, 4096).reshape(2 * EPAD1, C)
    xl1B = xl1[:, CA:]

    numA1, den1, w1 = _LOGITS_K2(src2g2, dst2g2, dst2p, xl1, xr1, e1, att1)
    numB1 = _APPLY_K2(src2g2, dst2p, xl1B, w1)
    h = _div1(numA1, numB1, den1.reshape(NCORE, NT, NPAD), b1.reshape(1, 2 * C))

    xl2 = _mm_headed(h, Wl2, 1, 1280).reshape(NPAD, C)
    xr2 = _mm_headed(h, Wr2, 1, 1280).reshape(NPAD, C)
    e2 = _mm_headed(ea_full, We2, 1, 4096).reshape(EPAD1, C)
    xl2B = xl2[:, CA:]

    numA2, den2, w2 = _LOGITS_K1(src2g1, dst2g1, dst2p, xl2, xr2, e2, att2)
    numB2 = _APPLY_K1(src2g1, dst2p, xl2B, w2)
    out = _div2(numA2, numB2, den2.reshape(NCORE, NT, NPAD), b2.reshape(1, C))
    return out[:N]
